# baseline (speedup 1.0000x reference)
import functools

import jax
import jax.numpy as jnp
from jax import lax
from jax.experimental import pallas as pl
from jax.experimental.pallas import tpu as pltpu
from jax.experimental.pallas import tpu_sc as plsc

ALPHA = 1.0
BETA = 0.3
GAMMA = 0.5
DELTA = 0.2
EMAX = 4.0
LAM = 0.5

N = 10000
E = 320000
D = 128

NC = 2
NS = 16
NW = NC * NS
EPW = E // NW
CHUNK = 80
NCHUNK = EPW // CHUNK
GROUPS = CHUNK // 16


DW = D // 2


_GB = 10
_RB = N // _GB


def _dense_body(amp_ref, my_ref, mu_ref,
                s1_ref, s2_ref, s3_ref, s4_ref):
    i = pl.program_id(0)

    @pl.when(i == 0)
    def _once():
        a = amp_ref[...]
        e = 0.5 * a * a
        s1_ref[0, 0] = jnp.sum(e)
        s2_ref[0, 0] = jnp.sum(e * e)
        r = jnp.maximum(e - EMAX, 0.0)
        s3_ref[0, 0] = jnp.sum(r * r)
        s4_ref[0, 0] = 0.0

    dm = my_ref[...] - mu_ref[...]
    s4_ref[0, 0] += jnp.sum(dm * dm)


_scalar_spec = pl.BlockSpec((1, 1), lambda i: (0, 0),
                            memory_space=pltpu.SMEM)

_dense_call = pl.pallas_call(
    _dense_body,
    grid=(_GB,),
    in_specs=[
        pl.BlockSpec((N,), lambda i: (0,)),
        pl.BlockSpec((_RB, D), lambda i: (i, 0)),
        pl.BlockSpec((_RB, D), lambda i: (i, 0)),
    ],
    out_shape=(
        jax.ShapeDtypeStruct((1, 1), jnp.float32),
        jax.ShapeDtypeStruct((1, 1), jnp.float32),
        jax.ShapeDtypeStruct((1, 1), jnp.float32),
        jax.ShapeDtypeStruct((1, 1), jnp.float32),
    ),
    out_specs=(
        _scalar_spec, _scalar_spec, _scalar_spec, _scalar_spec,
    ),
)


def _edge_body(edge_hbm, cos_hbm, sin_hbm, pos_hbm,
               out_hbm,
               src_v, dst_v, cos_v, sin_v,
               rs0, rd0, rs1, rd1, st2,
               sem_s0, sem_d0, sem_s1, sem_d1):
    c = lax.axis_index("c")
    s = lax.axis_index("s")
    wid = s * NC + c
    base = wid * EPW

    pltpu.async_copy(edge_hbm.at[0, pl.ds(base, EPW)], src_v, sem_s0)
    pltpu.async_copy(edge_hbm.at[1, pl.ds(base, EPW)], dst_v, sem_d0)
    pltpu.async_copy(cos_hbm, cos_v, sem_s1)
    pltpu.async_copy(sin_hbm, sin_v, sem_d1)
    pltpu.make_async_copy(edge_hbm.at[0, pl.ds(base, EPW)], src_v,
                          sem_s0).wait()
    pltpu.make_async_copy(edge_hbm.at[1, pl.ds(base, EPW)], dst_v,
                          sem_d0).wait()

    bufs = ((rs0, rd0, sem_s0, sem_d0), (rs1, rd1, sem_s1, sem_d1))

    def fire(chunk_idx, b):
        rs, rd, ss, sd = bufs[b]
        off = chunk_idx * CHUNK
        pltpu.async_copy(pos_hbm.at[src_v.at[pl.ds(off, CHUNK)]], rs, ss)
        pltpu.async_copy(pos_hbm.at[dst_v.at[pl.ds(off, CHUNK)]], rd, sd)

    def wait(b):
        rs, rd, ss, sd = bufs[b]
        pltpu.make_async_copy(pos_hbm.at[pl.ds(0, CHUNK)], rs, ss).wait()
        pltpu.make_async_copy(pos_hbm.at[pl.ds(0, CHUNK)], rd, sd).wait()

    mask_hi = jnp.full((16,), -65536, jnp.int32)

    def process(chunk_idx, b, carry):
        rs, rd, _, _ = bufs[b]
        off = chunk_idx * CHUNK

        def phase_body(g, accp):
            og = off + g * 16
            si = src_v[pl.ds(og, 16)]
            di = dst_v[pl.ds(og, 16)]
            cs = plsc.load_gather(cos_v, [si])
            cd = plsc.load_gather(cos_v, [di])
            ss_ = plsc.load_gather(sin_v, [si])
            sd_ = plsc.load_gather(sin_v, [di])
            return accp + (cs * cd + ss_ * sd_)

        def row_body(t, carry):
            accs = list(carry)
            for e2 in range(2):
                row = 2 * t + e2
                for k in range(4):
                    ps = plsc.bitcast(rs[row, pl.ds(16 * k, 16)],
                                      jnp.bfloat16)
                    pd = plsc.bitcast(rd[row, pl.ds(16 * k, 16)],
                                      jnp.bfloat16)
                    dif = ps - pd
                    d2 = dif * dif
                    u = plsc.bitcast(d2, jnp.int32)
                    hi = plsc.bitcast(u & mask_hi, jnp.float32)
                    lo = plsc.bitcast(lax.shift_left(u, 16), jnp.float32)
                    accs[k % 2] = accs[k % 2] + hi
                    accs[2 + k % 2] = accs[2 + k % 2] + lo
            return tuple(accs)

        accp = lax.fori_loop(0, CHUNK // 16, phase_body, carry[0])
        accg = lax.fori_loop(0, CHUNK // 2, row_body, carry[1:])
        return (accp,) + accg

    zero = jnp.zeros((16,), jnp.float32)

    fire(0, 0)
    pltpu.make_async_copy(cos_hbm, cos_v, sem_s1).wait()
    pltpu.make_async_copy(sin_hbm, sin_v, sem_d1).wait()

    carry = (zero, zero, zero, zero, zero)

    def pair_body(t, carry):
        for b in range(2):
            chunk_idx = 2 * t + b
            fire(chunk_idx + 1, 1 - b)
            wait(b)
            carry = process(chunk_idx, b, carry)
        return carry

    carry = lax.fori_loop(0, (NCHUNK - 1) // 2, pair_body, carry)
    wait(0)
    carry = process(NCHUNK - 1, 0, carry)

    accp, a0, a1, a2, a3 = carry
    st2[0, :] = accp
    st2[1, :] = (a0 + a1) + (a2 + a3)
    pltpu.sync_copy(st2, out_hbm.at[wid])


_edge_call = functools.partial(
    pl.kernel,
    out_type=jax.ShapeDtypeStruct((NW, 2, 16), jnp.float32),
    mesh=plsc.VectorSubcoreMesh(core_axis_name="c", subcore_axis_name="s"),
    compiler_params=pltpu.CompilerParams(needs_layout_passes=False,
                                         use_tc_tiling_on_sc=False),
    scratch_types=[
        pltpu.VMEM((EPW,), jnp.int32),
        pltpu.VMEM((EPW,), jnp.int32),
        pltpu.VMEM((N,), jnp.float32),
        pltpu.VMEM((N,), jnp.float32),
        pltpu.VMEM((CHUNK, DW), jnp.int32),
        pltpu.VMEM((CHUNK, DW), jnp.int32),
        pltpu.VMEM((CHUNK, DW), jnp.int32),
        pltpu.VMEM((CHUNK, DW), jnp.int32),
        pltpu.VMEM((2, 16), jnp.float32),
        pltpu.SemaphoreType.DMA,
        pltpu.SemaphoreType.DMA,
        pltpu.SemaphoreType.DMA,
        pltpu.SemaphoreType.DMA,
    ],
)(_edge_body)


def kernel(phi, amplitude, positions, edge_index, memory_y, memory_u):
    edges = edge_index.astype(jnp.int32)

    cos_t = jnp.cos(phi)
    sin_t = jnp.sin(phi)
    packed = lax.bitcast_convert_type(
        positions.astype(jnp.bfloat16).reshape(N, DW, 2), jnp.int32)

    s1, s2, s3, s4 = _dense_call(amplitude, memory_y, memory_u)

    outpg = _edge_call(edges, cos_t, sin_t, packed)

    sums = jnp.sum(outpg, axis=(0, 2))
    sum_cos = sums[0]
    sum_g = sums[1]
    l_phase = 1.0 - sum_cos / E
    l_graph = sum_g / E
    mean_e = s1[0, 0] / N
    l_energy = s2[0, 0] / N - mean_e * mean_e + LAM * (s3[0, 0] / N)
    l_mem = s4[0, 0] / (N * D)
    return (ALPHA * l_phase + BETA * l_energy
            + GAMMA * l_mem + DELTA * l_graph)


# speedup vs baseline: 1.1482x; 1.1482x over previous
import functools

import jax
import jax.numpy as jnp
from jax import lax
from jax.experimental import pallas as pl
from jax.experimental.pallas import tpu as pltpu
from jax.experimental.pallas import tpu_sc as plsc

ALPHA = 1.0
BETA = 0.3
GAMMA = 0.5
DELTA = 0.2
EMAX = 4.0
LAM = 0.5

N = 10000
E = 320000
D = 128

NC = 2
NS = 16
NW = NC * NS
EPW = E // NW
CHUNK = 80
NCHUNK = EPW // CHUNK
GROUPS = CHUNK // 16


DW = D // 2


def _rne_pack(a, b):
    ua = lax.bitcast_convert_type(a, jnp.uint32)
    ub = lax.bitcast_convert_type(b, jnp.uint32)
    ra = ua + jnp.uint32(0x7FFF) + ((ua >> 16) & jnp.uint32(1))
    rb = ub + jnp.uint32(0x7FFF) + ((ub >> 16) & jnp.uint32(1))
    w = (ra >> 16) | (rb & jnp.uint32(0xFFFF0000))
    return lax.bitcast_convert_type(w, jnp.int32)


_GB = 10
_RB = N // _GB


def _dense_body(phi_ref, amp_ref, my_ref, mu_ref, pos_ref,
                cos_ref, sin_ref, packed_ref,
                s1_ref, s2_ref, s3_ref, s4_ref):
    i = pl.program_id(0)

    @pl.when(i == 0)
    def _once():
        p = phi_ref[...]
        cos_ref[...] = jnp.cos(p)
        sin_ref[...] = jnp.sin(p)
        a = amp_ref[...]
        e = 0.5 * a * a
        s1_ref[0, 0] = jnp.sum(e)
        s2_ref[0, 0] = jnp.sum(e * e)
        r = jnp.maximum(e - EMAX, 0.0)
        s3_ref[0, 0] = jnp.sum(r * r)
        s4_ref[0, 0] = 0.0

    dm = my_ref[...] - mu_ref[...]
    s4_ref[0, 0] += jnp.sum(dm * dm)
    packed_ref[...] = _rne_pack(pos_ref[:, :DW], pos_ref[:, DW:])


_scalar_spec = pl.BlockSpec((1, 1), lambda i: (0, 0),
                            memory_space=pltpu.SMEM)

_dense_call = pl.pallas_call(
    _dense_body,
    grid=(_GB,),
    in_specs=[
        pl.BlockSpec((N,), lambda i: (0,)),
        pl.BlockSpec((N,), lambda i: (0,)),
        pl.BlockSpec((_RB, D), lambda i: (i, 0)),
        pl.BlockSpec((_RB, D), lambda i: (i, 0)),
        pl.BlockSpec((_RB, D), lambda i: (i, 0)),
    ],
    out_shape=(
        jax.ShapeDtypeStruct((N,), jnp.float32),
        jax.ShapeDtypeStruct((N,), jnp.float32),
        jax.ShapeDtypeStruct((N, DW), jnp.int32),
        jax.ShapeDtypeStruct((1, 1), jnp.float32),
        jax.ShapeDtypeStruct((1, 1), jnp.float32),
        jax.ShapeDtypeStruct((1, 1), jnp.float32),
        jax.ShapeDtypeStruct((1, 1), jnp.float32),
    ),
    out_specs=(
        pl.BlockSpec((N,), lambda i: (0,)),
        pl.BlockSpec((N,), lambda i: (0,)),
        pl.BlockSpec((_RB, DW), lambda i: (i, 0)),
        _scalar_spec, _scalar_spec, _scalar_spec, _scalar_spec,
    ),
)


def _edge_body(edge_hbm, cos_hbm, sin_hbm, pos_hbm,
               out_hbm,
               src_v, dst_v, cos_v, sin_v,
               rs0, rd0, rs1, rd1, st2,
               sem_s0, sem_d0, sem_s1, sem_d1):
    c = lax.axis_index("c")
    s = lax.axis_index("s")
    wid = s * NC + c
    base = wid * EPW

    pltpu.async_copy(edge_hbm.at[0, pl.ds(base, EPW)], src_v, sem_s0)
    pltpu.async_copy(edge_hbm.at[1, pl.ds(base, EPW)], dst_v, sem_d0)
    pltpu.async_copy(cos_hbm, cos_v, sem_s1)
    pltpu.async_copy(sin_hbm, sin_v, sem_d1)
    pltpu.make_async_copy(edge_hbm.at[0, pl.ds(base, EPW)], src_v,
                          sem_s0).wait()
    pltpu.make_async_copy(edge_hbm.at[1, pl.ds(base, EPW)], dst_v,
                          sem_d0).wait()

    bufs = ((rs0, rd0, sem_s0, sem_d0), (rs1, rd1, sem_s1, sem_d1))

    def fire(chunk_idx, b):
        rs, rd, ss, sd = bufs[b]
        off = chunk_idx * CHUNK
        pltpu.async_copy(pos_hbm.at[src_v.at[pl.ds(off, CHUNK)]], rs, ss)
        pltpu.async_copy(pos_hbm.at[dst_v.at[pl.ds(off, CHUNK)]], rd, sd)

    def wait(b):
        rs, rd, ss, sd = bufs[b]
        pltpu.make_async_copy(pos_hbm.at[pl.ds(0, CHUNK)], rs, ss).wait()
        pltpu.make_async_copy(pos_hbm.at[pl.ds(0, CHUNK)], rd, sd).wait()

    mask_hi = jnp.full((16,), -65536, jnp.int32)

    def process(chunk_idx, b, carry):
        rs, rd, _, _ = bufs[b]
        off = chunk_idx * CHUNK

        def phase_body(g, accp):
            og = off + g * 16
            si = src_v[pl.ds(og, 16)]
            di = dst_v[pl.ds(og, 16)]
            cs = plsc.load_gather(cos_v, [si])
            cd = plsc.load_gather(cos_v, [di])
            ss_ = plsc.load_gather(sin_v, [si])
            sd_ = plsc.load_gather(sin_v, [di])
            return accp + (cs * cd + ss_ * sd_)

        def row_body(t, carry):
            accs = list(carry)
            for e2 in range(2):
                row = 2 * t + e2
                for k in range(4):
                    ps = plsc.bitcast(rs[row, pl.ds(16 * k, 16)],
                                      jnp.bfloat16)
                    pd = plsc.bitcast(rd[row, pl.ds(16 * k, 16)],
                                      jnp.bfloat16)
                    dif = ps - pd
                    d2 = dif * dif
                    u = plsc.bitcast(d2, jnp.int32)
                    hi = plsc.bitcast(u & mask_hi, jnp.float32)
                    lo = plsc.bitcast(lax.shift_left(u, 16), jnp.float32)
                    accs[k % 2] = accs[k % 2] + hi
                    accs[2 + k % 2] = accs[2 + k % 2] + lo
            return tuple(accs)

        accp = lax.fori_loop(0, CHUNK // 16, phase_body, carry[0])
        accg = lax.fori_loop(0, CHUNK // 2, row_body, carry[1:])
        return (accp,) + accg

    zero = jnp.zeros((16,), jnp.float32)

    fire(0, 0)
    pltpu.make_async_copy(cos_hbm, cos_v, sem_s1).wait()
    pltpu.make_async_copy(sin_hbm, sin_v, sem_d1).wait()

    carry = (zero, zero, zero, zero, zero)

    def pair_body(t, carry):
        for b in range(2):
            chunk_idx = 2 * t + b
            fire(chunk_idx + 1, 1 - b)
            wait(b)
            carry = process(chunk_idx, b, carry)
        return carry

    carry = lax.fori_loop(0, (NCHUNK - 1) // 2, pair_body, carry)
    wait(0)
    carry = process(NCHUNK - 1, 0, carry)

    accp, a0, a1, a2, a3 = carry
    st2[0, :] = accp
    st2[1, :] = (a0 + a1) + (a2 + a3)
    pltpu.sync_copy(st2, out_hbm.at[wid])


_edge_call = functools.partial(
    pl.kernel,
    out_type=jax.ShapeDtypeStruct((NW, 2, 16), jnp.float32),
    mesh=plsc.VectorSubcoreMesh(core_axis_name="c", subcore_axis_name="s"),
    compiler_params=pltpu.CompilerParams(needs_layout_passes=False,
                                         use_tc_tiling_on_sc=False),
    scratch_types=[
        pltpu.VMEM((EPW,), jnp.int32),
        pltpu.VMEM((EPW,), jnp.int32),
        pltpu.VMEM((N,), jnp.float32),
        pltpu.VMEM((N,), jnp.float32),
        pltpu.VMEM((CHUNK, DW), jnp.int32),
        pltpu.VMEM((CHUNK, DW), jnp.int32),
        pltpu.VMEM((CHUNK, DW), jnp.int32),
        pltpu.VMEM((CHUNK, DW), jnp.int32),
        pltpu.VMEM((2, 16), jnp.float32),
        pltpu.SemaphoreType.DMA,
        pltpu.SemaphoreType.DMA,
        pltpu.SemaphoreType.DMA,
        pltpu.SemaphoreType.DMA,
    ],
)(_edge_body)


def kernel(phi, amplitude, positions, edge_index, memory_y, memory_u):
    edges = edge_index.astype(jnp.int32)

    cos_t, sin_t, packed, s1, s2, s3, s4 = _dense_call(
        phi, amplitude, memory_y, memory_u, positions)

    outpg = _edge_call(edges, cos_t, sin_t, packed)

    sums = jnp.sum(outpg, axis=(0, 2))
    sum_cos = sums[0]
    sum_g = sums[1]
    l_phase = 1.0 - sum_cos / E
    l_graph = sum_g / E
    mean_e = s1[0, 0] / N
    l_energy = s2[0, 0] / N - mean_e * mean_e + LAM * (s3[0, 0] / N)
    l_mem = s4[0, 0] / (N * D)
    return (ALPHA * l_phase + BETA * l_energy
            + GAMMA * l_mem + DELTA * l_graph)
